# baseline (speedup 1.0000x reference)
import functools

import jax
import jax.numpy as jnp
from jax import lax
from jax.experimental import pallas as pl
from jax.experimental.pallas import tpu as pltpu
from jax.experimental.pallas import tpu_sc as plsc

_D = 768
_T = 77
_B = 4096
_BT = _B * _T

_NC = 2
_NS = 16
_NW = _NC * _NS
_SAMP_W = _B // _NW
_PER_W = _BT // _NW
_CHUNK = 16
_NCC = _SAMP_W // _CHUNK
_NCH = _T * _NCC
_NBUF = 4
_LANES = 16
_ND = _D // _LANES


def _sc_embed(tokens_flat, table, pos):
    mesh = plsc.VectorSubcoreMesh(core_axis_name="c", subcore_axis_name="s")

    @functools.partial(
        pl.kernel,
        out_type=jax.ShapeDtypeStruct((_BT, _D), jnp.float32),
        mesh=mesh,
        scratch_types=[
            pltpu.VMEM((_NBUF, _CHUNK), jnp.int32),
            pltpu.VMEM((_T, _D), jnp.float32),
        ]
        + [pltpu.VMEM((_CHUNK, _D), jnp.float32)] * _NBUF
        + [pltpu.SemaphoreType.DMA] * (3 * _NBUF),
    )
    def run(tok_hbm, tab_hbm, pos_hbm, out_hbm, cidx_v, pos_v, *rest):
        bufs = rest[:_NBUF]
        gsems = rest[_NBUF:2 * _NBUF]
        ssems = rest[2 * _NBUF:3 * _NBUF]
        csems = rest[3 * _NBUF:]

        wid = lax.axis_index("s") * _NC + lax.axis_index("c")
        base = wid * _PER_W
        pltpu.sync_copy(pos_hbm, pos_v)

        iota77 = lax.iota(jnp.int32, _LANES) * _T

        def chunk_t_off(j):
            t = j // _NCC
            c = lax.rem(j, _NCC)
            return t, c * (_CHUNK * _T) + t

        def start_idx_fetch(j, slot):
            _, off = chunk_t_off(j)
            rvec = iota77 + (base + off)
            pltpu.async_copy(
                tok_hbm.at[rvec], cidx_v.at[slot], csems[slot]
            )

        def wait_idx_fetch(slot):
            pltpu.make_async_copy(
                tok_hbm.at[pl.ds(0, _CHUNK)], cidx_v.at[slot], csems[slot]
            ).wait()

        def start_table_gather(slot):
            pltpu.async_copy(
                tab_hbm.at[cidx_v.at[slot]], bufs[slot], gsems[slot]
            )

        def wait_scatter(slot):
            pltpu.make_async_copy(
                bufs[slot], out_hbm.at[pl.ds(0, _CHUNK)], ssems[slot]
            ).wait()

        for s in range(_NBUF):
            start_idx_fetch(s, s)
        for s in range(2):
            wait_idx_fetch(s)
            start_table_gather(s)

        def outer(i, carry):
            for b in range(_NBUF):
                j = i * _NBUF + b
                nslot = (b + 2) % _NBUF

                @pl.when(j + 2 < _NCH)
                def _():
                    wait_idx_fetch(nslot)

                    @pl.when(j >= 2)
                    def _():
                        wait_scatter(nslot)

                    start_table_gather(nslot)

                pltpu.make_async_copy(
                    tab_hbm.at[pl.ds(0, _CHUNK)], bufs[b], gsems[b]
                ).wait()

                @pl.when(j + 4 < _NCH)
                def _():
                    start_idx_fetch(j + 4, b)

                t, off = chunk_t_off(j)
                buf = bufs[b]

                @plsc.parallel_loop(0, _ND, 1, unroll=2)
                def dcol(dblk):
                    sl = pl.ds(dblk * _LANES, _LANES)
                    p = pos_v[t, sl]
                    for r in range(_CHUNK):
                        plsc.addupdate(buf.at[r, sl], p)

                ovec = iota77 + (base + off)
                pltpu.async_copy(buf, out_hbm.at[ovec], ssems[b])
            return carry

        lax.fori_loop(0, _NCH // _NBUF, outer, 0)

        for b in range(_NBUF):
            wait_scatter(b)

    return run(tokens_flat, table, pos)


def kernel(tokens, token_embedding, position_embedding):
    idx = tokens.reshape(-1).astype(jnp.int32)
    out = _sc_embed(idx, token_embedding, position_embedding)
    return out.reshape(_B, _T, _D)


# speedup vs baseline: 1.5116x; 1.5116x over previous
import functools

import jax
import jax.numpy as jnp
from jax import lax
from jax.experimental import pallas as pl
from jax.experimental.pallas import tpu as pltpu
from jax.experimental.pallas import tpu_sc as plsc

_D = 768
_T = 77
_B = 4096
_BT = _B * _T

_NC = 2
_NS = 16
_NW = _NC * _NS
_SAMP_W = _B // _NW
_PER_W = _BT // _NW
_TP = 80
_CHUNK = 16
_NCC = _SAMP_W // _CHUNK
_NCH = _T * _NCC
_NBUF = 4
_LANES = 16
_ND = _D // _LANES


def _sc_embed(tokens_flat, table, pos):
    mesh = plsc.VectorSubcoreMesh(core_axis_name="c", subcore_axis_name="s")

    @functools.partial(
        pl.kernel,
        out_type=jax.ShapeDtypeStruct((_B * _TP, _D), jnp.float32),
        mesh=mesh,
        scratch_types=[
            pltpu.VMEM((_NBUF, _CHUNK), jnp.int32),
            pltpu.VMEM((_T, _D), jnp.float32),
        ]
        + [pltpu.VMEM((_CHUNK, _D), jnp.float32)] * _NBUF
        + [pltpu.SemaphoreType.DMA] * (3 * _NBUF),
    )
    def run(tok_hbm, tab_hbm, pos_hbm, out_hbm, cidx_v, pos_v, *rest):
        bufs = rest[:_NBUF]
        gsems = rest[_NBUF:2 * _NBUF]
        ssems = rest[2 * _NBUF:3 * _NBUF]
        csems = rest[3 * _NBUF:]

        wid = lax.axis_index("s") * _NC + lax.axis_index("c")
        base = wid * _PER_W
        pltpu.sync_copy(pos_hbm, pos_v)

        iota77 = lax.iota(jnp.int32, _LANES) * _T
        iota80 = lax.iota(jnp.int32, _LANES) * _TP

        def chunk_t_off(j):
            t = j // _NCC
            c = lax.rem(j, _NCC)
            return t, c * (_CHUNK * _T) + t, c * (_CHUNK * _TP) + t

        def start_idx_fetch(j, slot):
            _, off, _ = chunk_t_off(j)
            rvec = iota77 + (base + off)
            pltpu.async_copy(
                tok_hbm.at[rvec], cidx_v.at[slot], csems[slot]
            )

        def wait_idx_fetch(slot):
            pltpu.make_async_copy(
                tok_hbm.at[pl.ds(0, _CHUNK)], cidx_v.at[slot], csems[slot]
            ).wait()

        def start_table_gather(slot):
            pltpu.async_copy(
                tab_hbm.at[cidx_v.at[slot]], bufs[slot], gsems[slot]
            )

        def wait_scatter(slot):
            pltpu.make_async_copy(
                bufs[slot], out_hbm.at[pl.ds(0, _CHUNK)], ssems[slot]
            ).wait()

        for s in range(_NBUF):
            start_idx_fetch(s, s)
        for s in range(2):
            wait_idx_fetch(s)
            start_table_gather(s)

        def outer(i, carry):
            for b in range(_NBUF):
                j = i * _NBUF + b
                nslot = (b + 2) % _NBUF

                @pl.when(j + 2 < _NCH)
                def _():
                    wait_idx_fetch(nslot)

                    @pl.when(j >= 2)
                    def _():
                        wait_scatter(nslot)

                    start_table_gather(nslot)

                pltpu.make_async_copy(
                    tab_hbm.at[pl.ds(0, _CHUNK)], bufs[b], gsems[b]
                ).wait()

                @pl.when(j + 4 < _NCH)
                def _():
                    start_idx_fetch(j + 4, b)

                t, _, ooff = chunk_t_off(j)
                buf = bufs[b]

                @plsc.parallel_loop(0, _ND, 1, unroll=2)
                def dcol(dblk):
                    sl = pl.ds(dblk * _LANES, _LANES)
                    p = pos_v[t, sl]
                    for r in range(_CHUNK):
                        plsc.addupdate(buf.at[r, sl], p)

                ovec = iota80 + (wid * _SAMP_W * _TP + ooff)
                pltpu.async_copy(buf, out_hbm.at[ovec], ssems[b])
            return carry

        lax.fori_loop(0, _NCH // _NBUF, outer, 0)

        for b in range(_NBUF):
            wait_scatter(b)

    return run(tokens_flat, table, pos)


def kernel(tokens, token_embedding, position_embedding):
    idx = tokens.reshape(-1).astype(jnp.int32)
    out = _sc_embed(idx, token_embedding, position_embedding)
    return out.reshape(_B, _TP, _D)[:, :_T, :]


# speedup vs baseline: 1.5374x; 1.0171x over previous
import functools

import jax
import jax.numpy as jnp
from jax import lax
from jax.experimental import pallas as pl
from jax.experimental.pallas import tpu as pltpu
from jax.experimental.pallas import tpu_sc as plsc

_D = 768
_T = 77
_B = 4096

_NC = 2
_NS = 16
_NW = _NC * _NS
_SAMP_W = _B // _NW
_NTB = 9
_TT = 72
_TTW = _T - _TT
_CHUNK = 16
_NBUF = 4
_NPAIR = _SAMP_W // 2
_NCH1 = _NPAIR * _NTB
_NCH2 = _NPAIR
_LANES = 16
_ND = _D // _LANES


def _sc_embed(tokens_flat, table, pos):
    mesh = plsc.VectorSubcoreMesh(core_axis_name="c", subcore_axis_name="s")

    @functools.partial(
        pl.kernel,
        out_type=jax.ShapeDtypeStruct((_B, _T, _D), jnp.float32),
        mesh=mesh,
        scratch_types=[
            pltpu.VMEM((_NBUF, _CHUNK), jnp.int32),
            pltpu.VMEM((_T, _D), jnp.float32),
        ]
        + [pltpu.VMEM((_CHUNK, _D), jnp.float32)] * _NBUF
        + [pltpu.SemaphoreType.DMA] * (3 * _NBUF),
    )
    def run(tok_hbm, tab_hbm, pos_hbm, out_hbm, cidx_v, pos_v, *rest):
        bufs = rest[:_NBUF]
        gsems = rest[_NBUF:2 * _NBUF]
        ssems = rest[2 * _NBUF:3 * _NBUF]
        csems = rest[3 * _NBUF:]

        wid = lax.axis_index("s") * _NC + lax.axis_index("c")
        b_lo = wid * _SAMP_W
        base = b_lo * _T
        pltpu.sync_copy(pos_hbm, pos_v)

        iota = lax.iota(jnp.int32, _LANES)
        pat_main = (iota >> 3) * _T + (iota & 7)
        pat_tail = (iota >> 3) * _T + jnp.minimum(iota & 7, _TTW - 1) + _TT

        def start_idx_fetch(rvec, slot):
            pltpu.async_copy(tok_hbm.at[rvec], cidx_v.at[slot], csems[slot])

        def wait_idx_fetch(slot):
            pltpu.make_async_copy(
                tok_hbm.at[pl.ds(0, _CHUNK)], cidx_v.at[slot], csems[slot]
            ).wait()

        def start_table_gather(slot):
            pltpu.async_copy(
                tab_hbm.at[cidx_v.at[slot]], bufs[slot], gsems[slot]
            )

        def wait_table_gather(slot):
            pltpu.make_async_copy(
                tab_hbm.at[pl.ds(0, _CHUNK)], bufs[slot], gsems[slot]
            ).wait()

        def wait_scatter(slot, width):
            for _ in range(2):
                pltpu.make_async_copy(
                    bufs[slot].at[pl.ds(0, width)],
                    out_hbm.at[0, pl.ds(0, width), :],
                    ssems[slot],
                ).wait()

        def add_pos(buf, t0, width):
            @plsc.parallel_loop(0, _ND, 1, unroll=2)
            def dcol(dblk):
                sl = pl.ds(dblk * _LANES, _LANES)
                for tt in range(width):
                    p = pos_v[t0 + tt, sl]
                    plsc.addupdate(buf.at[tt, sl], p)
                    plsc.addupdate(buf.at[8 + tt, sl], p)

        def scatter(buf, bb, t0, width, slot):
            pltpu.async_copy(
                buf.at[pl.ds(0, width)],
                out_hbm.at[bb, pl.ds(t0, width), :],
                ssems[slot],
            )
            pltpu.async_copy(
                buf.at[pl.ds(8, width)],
                out_hbm.at[bb + 1, pl.ds(t0, width), :],
                ssems[slot],
            )

        def rvec1(j):
            sp = j // _NTB
            tb = j - sp * _NTB
            return pat_main + (base + sp * (2 * _T) + tb * 8), sp, tb

        for s in range(_NBUF):
            start_idx_fetch(rvec1(s)[0], s)
        for s in range(2):
            wait_idx_fetch(s)
            start_table_gather(s)

        def outer1(i, carry):
            for b in range(_NBUF):
                j = i * _NBUF + b
                nslot = (b + 2) % _NBUF

                @pl.when(j + 2 < _NCH1)
                def _():
                    wait_idx_fetch(nslot)

                    @pl.when(j >= 2)
                    def _():
                        wait_scatter(nslot, 8)

                    start_table_gather(nslot)

                wait_table_gather(b)

                @pl.when(j + 4 < _NCH1)
                def _():
                    start_idx_fetch(rvec1(j + 4)[0], b)

                _, sp, tb = rvec1(j)
                t0 = pl.multiple_of(tb * 8, 8)
                add_pos(bufs[b], t0, 8)
                scatter(bufs[b], b_lo + 2 * sp, t0, 8, b)
            return carry

        lax.fori_loop(0, _NCH1 // _NBUF, outer1, 0)
        for b in range(_NBUF):
            wait_scatter(b, 8)

        def rvec2(k):
            return pat_tail + (base + k * (2 * _T))

        for s in range(_NBUF):
            start_idx_fetch(rvec2(s), s)
        for s in range(2):
            wait_idx_fetch(s)
            start_table_gather(s)

        def outer2(i, carry):
            for b in range(_NBUF):
                k = i * _NBUF + b
                nslot = (b + 2) % _NBUF

                @pl.when(k + 2 < _NCH2)
                def _():
                    wait_idx_fetch(nslot)

                    @pl.when(k >= 2)
                    def _():
                        wait_scatter(nslot, _TTW)

                    start_table_gather(nslot)

                wait_table_gather(b)

                @pl.when(k + 4 < _NCH2)
                def _():
                    start_idx_fetch(rvec2(k + 4), b)

                add_pos(bufs[b], _TT, _TTW)
                scatter(bufs[b], b_lo + 2 * k, _TT, _TTW, b)
            return carry

        lax.fori_loop(0, _NCH2 // _NBUF, outer2, 0)
        for b in range(_NBUF):
            wait_scatter(b, _TTW)

    return run(tokens_flat, table, pos)


def kernel(tokens, token_embedding, position_embedding):
    idx = tokens.reshape(-1).astype(jnp.int32)
    return _sc_embed(idx, token_embedding, position_embedding)
